# baseline (speedup 1.0000x reference)
import functools

import jax
import jax.numpy as jnp
from jax import lax
from jax.experimental import pallas as pl
from jax.experimental.pallas import tpu as pltpu
from jax.experimental.pallas import tpu_sc as plsc

N_EXP = 8
D = 768
N_TOK = 32768
NW = 32
TOK_PER_W = N_TOK // NW
LANES = 16
GROUPS = TOK_PER_W // LANES


def _proj_body(w_ref, x_ref, o_ref):
    o_ref[0] = lax.dot_general(
        w_ref[...], x_ref[...],
        dimension_numbers=(((1,), (1,)), ((), ())),
        preferred_element_type=jnp.float32,
    )


def _project(tokens, W):
    return pl.pallas_call(
        _proj_body,
        grid=(NW,),
        in_specs=[
            pl.BlockSpec((N_EXP, D), lambda i: (0, 0)),
            pl.BlockSpec((TOK_PER_W, D), lambda i: (i, 0)),
        ],
        out_specs=pl.BlockSpec((1, N_EXP, TOK_PER_W), lambda i: (i, 0, 0)),
        out_shape=jax.ShapeDtypeStruct((NW, N_EXP, TOK_PER_W), jnp.float32),
    )(W, tokens)


_mesh = plsc.VectorSubcoreMesh(core_axis_name="c", subcore_axis_name="s")


@functools.partial(
    pl.kernel,
    mesh=_mesh,
    out_type=[
        jax.ShapeDtypeStruct((2, N_TOK), jnp.float32),
        jax.ShapeDtypeStruct((2, N_TOK), jnp.int32),
    ],
    scratch_types=[
        pltpu.VMEM((N_EXP, TOK_PER_W), jnp.float32),
        pltpu.VMEM((TOK_PER_W,), jnp.float32),
        pltpu.VMEM((TOK_PER_W,), jnp.float32),
        pltpu.VMEM((TOK_PER_W,), jnp.int32),
        pltpu.VMEM((TOK_PER_W,), jnp.int32),
    ],
)
def _route(lg_hbm, sc_hbm, ix_hbm, lg_v, s1_v, s2_v, i1_v, i2_v):
    wid = lax.axis_index("s") * 2 + lax.axis_index("c")
    pltpu.sync_copy(lg_hbm.at[wid], lg_v)

    def body(g, carry):
        base = g * LANES
        vs = [lg_v[e, pl.ds(base, LANES)] for e in range(N_EXP)]
        m = vs[0]
        for e in range(1, N_EXP):
            m = jnp.maximum(m, vs[e])
        ex = [jnp.exp(vs[e] - m) for e in range(N_EXP)]
        tot = ex[0]
        for e in range(1, N_EXP):
            tot = tot + ex[e]
        sx = [ex[e] / tot for e in range(N_EXP)]
        v1 = sx[0]
        i1 = jnp.zeros((LANES,), jnp.int32)
        for e in range(1, N_EXP):
            gt = sx[e] > v1
            v1 = jnp.where(gt, sx[e], v1)
            i1 = jnp.where(gt, jnp.int32(e), i1)
        v2 = jnp.full((LANES,), -1.0, jnp.float32)
        i2 = jnp.zeros((LANES,), jnp.int32)
        for e in range(N_EXP):
            ok = (sx[e] > v2) & (i1 != jnp.int32(e))
            v2 = jnp.where(ok, sx[e], v2)
            i2 = jnp.where(ok, jnp.int32(e), i2)
        s1_v[pl.ds(base, LANES)] = v1
        s2_v[pl.ds(base, LANES)] = v2
        i1_v[pl.ds(base, LANES)] = i1
        i2_v[pl.ds(base, LANES)] = i2
        return carry

    lax.fori_loop(0, GROUPS, body, 0)

    out0 = wid * TOK_PER_W
    pltpu.sync_copy(s1_v, sc_hbm.at[0, pl.ds(out0, TOK_PER_W)])
    pltpu.sync_copy(s2_v, sc_hbm.at[1, pl.ds(out0, TOK_PER_W)])
    pltpu.sync_copy(i1_v, ix_hbm.at[0, pl.ds(out0, TOK_PER_W)])
    pltpu.sync_copy(i2_v, ix_hbm.at[1, pl.ds(out0, TOK_PER_W)])


def kernel(tokens, W):
    logits3 = _project(tokens, W)
    scores, idx = _route(logits3)
    return (jnp.stack([scores[0], scores[1]], axis=1),
            jnp.stack([idx[0], idx[1]], axis=1))
